# baseline (speedup 1.0000x reference)
import jax
import jax.numpy as jnp
from jax.experimental import pallas as pl
from jax.experimental.pallas import tpu as pltpu

_NUM_NODES = 64
_D_FEAT = 256
_D_HID = 1024
_D_OUT = 1024
_BATCH = 4096
_CPANEL = 512
_RBLOCK = 1024


def _mlp_bcast_kernel(emb_ref, w1_ref, b1_ref, w2_ref, b2_ref, out_ref,
                      hbar_ref, g_ref):
    j = pl.program_id(0)
    i = pl.program_id(1)

    @pl.when(jnp.logical_and(j == 0, i == 0))
    def _compute_hbar():
        h = jnp.dot(emb_ref[:], w1_ref[:], preferred_element_type=jnp.float32)
        h = jnp.maximum(h + b1_ref[:], 0.0)
        hbar_ref[:] = jnp.mean(h, axis=0, keepdims=True)

    @pl.when(i == 0)
    def _compute_g_panel():
        g = jnp.dot(hbar_ref[:], w2_ref[:], preferred_element_type=jnp.float32)
        g_ref[:] = g + b2_ref[:]

    out_ref[:] = jnp.broadcast_to(g_ref[:], (_RBLOCK, _CPANEL))


def kernel(emb, W1, b1, W2, b2, batch_size):
    del batch_size
    b1r = b1.reshape(1, _D_HID)
    b2r = b2.reshape(1, _D_OUT)
    grid = (_D_OUT // _CPANEL, _BATCH // _RBLOCK)
    return pl.pallas_call(
        _mlp_bcast_kernel,
        grid=grid,
        in_specs=[
            pl.BlockSpec((_NUM_NODES, _D_FEAT), lambda j, i: (0, 0)),
            pl.BlockSpec((_D_FEAT, _D_HID), lambda j, i: (0, 0)),
            pl.BlockSpec((1, _D_HID), lambda j, i: (0, 0)),
            pl.BlockSpec((_D_HID, _CPANEL), lambda j, i: (0, j)),
            pl.BlockSpec((1, _CPANEL), lambda j, i: (0, j)),
        ],
        out_specs=pl.BlockSpec((_RBLOCK, _CPANEL), lambda j, i: (i, j)),
        out_shape=jax.ShapeDtypeStruct((_BATCH, _D_OUT), jnp.float32),
        scratch_shapes=[
            pltpu.VMEM((1, _D_HID), jnp.float32),
            pltpu.VMEM((1, _CPANEL), jnp.float32),
        ],
    )(emb, W1, b1r, W2, b2r)


# speedup vs baseline: 1.2574x; 1.2574x over previous
import jax
import jax.numpy as jnp
from jax.experimental import pallas as pl
from jax.experimental.pallas import tpu as pltpu

_NUM_NODES = 64
_D_FEAT = 256
_D_HID = 1024
_D_OUT = 1024
_BATCH = 4096
_CP = 512
_NP = _D_OUT // _CP
_RB = 1024
_NR = _BATCH // _RB


def _body(emb_ref, w1_ref, b1_ref, w2_hbm, b2_ref, out_hbm,
          w2v, bcv, sem_w2, sem_out):
    w2_copies = []
    for p in range(_NP):
        c = pltpu.make_async_copy(
            w2_hbm.at[:, pl.ds(p * _CP, _CP)], w2v.at[p], sem_w2.at[p])
        c.start()
        w2_copies.append(c)

    h = jnp.dot(emb_ref[...], w1_ref[...], preferred_element_type=jnp.float32)
    h = jnp.maximum(h + b1_ref[...], 0.0)
    hbar = jnp.mean(h, axis=0, keepdims=True)

    out_copies = []
    for p in range(_NP):
        w2_copies[p].wait()
        g = jnp.dot(hbar, w2v[p], preferred_element_type=jnp.float32)
        g = g + b2_ref[:, p * _CP:(p + 1) * _CP]
        bcv[p] = jnp.broadcast_to(g, (_RB, _CP))
        for i in range(_NR):
            c = pltpu.make_async_copy(
                bcv.at[p],
                out_hbm.at[pl.ds(i * _RB, _RB), pl.ds(p * _CP, _CP)],
                sem_out.at[p * _NR + i])
            c.start()
            out_copies.append(c)

    for c in out_copies:
        c.wait()


def kernel(emb, W1, b1, W2, b2, batch_size):
    del batch_size
    b1r = b1.reshape(1, _D_HID)
    b2r = b2.reshape(1, _D_OUT)
    return pl.pallas_call(
        _body,
        in_specs=[
            pl.BlockSpec(memory_space=pltpu.VMEM),
            pl.BlockSpec(memory_space=pltpu.VMEM),
            pl.BlockSpec(memory_space=pltpu.VMEM),
            pl.BlockSpec(memory_space=pl.ANY),
            pl.BlockSpec(memory_space=pltpu.VMEM),
        ],
        out_specs=pl.BlockSpec(memory_space=pl.ANY),
        out_shape=jax.ShapeDtypeStruct((_BATCH, _D_OUT), jnp.float32),
        scratch_shapes=[
            pltpu.VMEM((_NP, _D_HID, _CP), jnp.float32),
            pltpu.VMEM((_NP, _RB, _CP), jnp.float32),
            pltpu.SemaphoreType.DMA((_NP,)),
            pltpu.SemaphoreType.DMA((_NP * _NR,)),
        ],
    )(emb, W1, b1r, W2, b2r)
